# baseline (speedup 1.0000x reference)
import functools

import jax
import jax.numpy as jnp
from jax import lax
from jax.experimental import pallas as pl
from jax.experimental.pallas import tpu as pltpu
from jax.experimental.pallas import tpu_sc as plsc

PAD = 56
GB = 2
WB = 8


@functools.cache
def _build(batch: int, hist: int, vocab: int, d: int):
  info = plsc.get_sparse_core_info()
  nc, ns = info.num_cores, info.num_subcores
  nw = nc * ns
  per_w = batch // nw
  steps = per_w // WB
  gpg = WB // GB
  assert batch == nw * per_w and per_w % WB == 0 and steps % 2 == 0

  mesh = plsc.VectorSubcoreMesh(core_axis_name="c", subcore_axis_name="s")

  nbuf = 4
  groups = per_w // GB
  gsteps = groups // nbuf

  def body(idx_hbm, table_hbm, out_hbm, idx_v, bufs, sems):
    wid = lax.axis_index("s") * nc + lax.axis_index("c")
    obase = wid * per_w

    pltpu.sync_copy(idx_hbm.at[pl.ds(wid * groups, groups)], idx_v)

    def gather(g, b):
      pltpu.async_copy(table_hbm.at[idx_v.at[g]], bufs[b], sems[b])

    def gwait(b):
      pltpu.make_async_copy(
          table_hbm.at[idx_v.at[0]], bufs[b], sems[b]).wait()

    for b in range(nbuf):
      gather(b, b)

    def step(i, carry):
      g0 = nbuf * i
      for b in range(nbuf):
        gwait(b)
        for j in range(GB):
          pltpu.sync_copy(
              bufs[b].at[pl.ds(j * PAD, hist)],
              out_hbm.at[obase + (g0 + b) * GB + j])

        @pl.when(i < gsteps - 1)
        def _(b=b):
          gather(g0 + nbuf + b, b)

      return carry

    lax.fori_loop(0, gsteps, step, 0)

  return pl.kernel(
      body,
      out_type=jax.ShapeDtypeStruct((batch, hist, d), jnp.float32),
      mesh=mesh,
      scratch_types=[
          pltpu.VMEM((batch // nw // GB, GB * PAD), jnp.int32),
          [pltpu.VMEM((GB * PAD, d), jnp.float32) for _ in range(nbuf)],
          [pltpu.SemaphoreType.DMA for _ in range(nbuf)],
      ],
  )


@jax.jit
def kernel(x, table):
  b, h = x.shape
  vocab, d = table.shape
  xp = jnp.pad(x.astype(jnp.int32), ((0, 0), (0, PAD - h)))
  idx = xp.reshape(b // GB, GB * PAD)
  return _build(b, h, vocab, d)(idx, table)


# speedup vs baseline: 7.4779x; 7.4779x over previous
import functools

import jax
import jax.numpy as jnp
from jax import lax
from jax.experimental import pallas as pl
from jax.experimental.pallas import tpu as pltpu
from jax.experimental.pallas import tpu_sc as plsc

PAD = 56
GB = 2
WB = 8


@functools.cache
def _build(batch: int, hist: int, vocab: int, d: int):
  info = plsc.get_sparse_core_info()
  nc, ns = info.num_cores, info.num_subcores
  nw = nc * ns
  per_w = batch // nw
  steps = per_w // WB
  gpg = WB // GB
  assert batch == nw * per_w and per_w % WB == 0 and steps % 2 == 0

  mesh = plsc.VectorSubcoreMesh(core_axis_name="c", subcore_axis_name="s")

  nbuf = 4
  groups = per_w // GB
  gsteps = groups // nbuf

  def body(idx_hbm, table_hbm, out_hbm, idx_v, bufs, sems):
    wid = lax.axis_index("s") * nc + lax.axis_index("c")
    obase = wid * per_w

    pltpu.sync_copy(idx_hbm.at[pl.ds(wid * groups, groups)], idx_v)

    def gather(g, b):
      pltpu.async_copy(table_hbm.at[idx_v.at[g]], bufs[b], sems[b])

    def gwait(b):
      pltpu.make_async_copy(
          table_hbm.at[idx_v.at[0]], bufs[b], sems[b]).wait()

    for b in range(nbuf):
      gather(b, b)

    def step(i, carry):
      g0 = nbuf * i
      for b in range(nbuf):
        gwait(b)
        for j in range(GB):
          pltpu.sync_copy(
              bufs[b].at[pl.ds(j * PAD, hist)],
              out_hbm.at[obase + (g0 + b) * GB + j])

        @pl.when(i < gsteps - 1)
        def _(b=b):
          gather(g0 + nbuf + b, b)

      return carry

    lax.fori_loop(0, gsteps, step, 0)

  return pl.kernel(
      body,
      out_type=jax.ShapeDtypeStruct((batch, hist, d), jnp.float32),
      mesh=mesh,
      scratch_types=[
          pltpu.VMEM((batch // nw // GB, GB * PAD), jnp.int32),
          [pltpu.VMEM((GB * PAD, d), jnp.float32) for _ in range(nbuf)],
          [pltpu.SemaphoreType.DMA for _ in range(nbuf)],
      ],
  )


@jax.jit
def kernel(x, table):
  b, h = x.shape
  vocab, d = table.shape
  fill = jnp.arange(b * (PAD - h), dtype=jnp.int32).reshape(b, PAD - h) % vocab
  xp = jnp.concatenate([x.astype(jnp.int32), fill], axis=1)
  idx = xp.reshape(b // GB, GB * PAD)
  return _build(b, h, vocab, d)(idx, table)
